# baseline (speedup 1.0000x reference)
import functools

import jax
import jax.numpy as jnp
from jax import lax
from jax.experimental import pallas as pl
from jax.experimental.pallas import tpu as pltpu
from jax.experimental.pallas import tpu_sc as plsc

HS = 128
HV = 128
FEAT = HS + 3 * HV
SEG = 16
OUT_F = HS + HV
NC, NSUB, L = 2, 16, 16
NW = NC * NSUB
CHUNK = 128


def _tc_norms(node_features):
    n = node_features.shape[0]
    blk = 2048
    steps = n // blk
    W = 3 * HV

    def body(hbm_ref, o_ref, buf, s0, s1):
        i = pl.program_id(0)

        def src(step):
            return hbm_ref.at[pl.ds(step * blk, blk), pl.ds(HS, W)]

        @pl.when(i == 0)
        def _():
            pltpu.async_copy(src(0), buf.at[0], s0)

        nxt = i + 1

        @pl.when(jnp.logical_and(nxt < steps, nxt % 2 == 0))
        def _():
            pltpu.async_copy(src(nxt), buf.at[0], s0)

        @pl.when(jnp.logical_and(nxt < steps, nxt % 2 == 1))
        def _():
            pltpu.async_copy(src(nxt), buf.at[1], s1)

        @pl.when(i % 2 == 0)
        def _():
            pltpu.make_async_copy(src(i), buf.at[0], s0).wait()

        @pl.when(i % 2 == 1)
        def _():
            pltpu.make_async_copy(src(i), buf.at[1], s1).wait()

        v = buf[i % 2]
        sq = v * v
        r = lax.broadcasted_iota(jnp.int32, (W, HV), 0)
        c = lax.broadcasted_iota(jnp.int32, (W, HV), 1)
        sel = jnp.where(r // 3 == c, 1.0, 0.0)
        ss = jnp.dot(sq, sel, preferred_element_type=jnp.float32)
        o_ref[...] = jnp.sqrt(ss)

    return pl.pallas_call(
        body,
        grid=(steps,),
        in_specs=[pl.BlockSpec(memory_space=pltpu.MemorySpace.HBM)],
        out_specs=pl.BlockSpec((blk, HV), lambda i: (i, 0)),
        out_shape=jax.ShapeDtypeStruct((n, HV), jnp.float32),
        scratch_shapes=[
            pltpu.VMEM((2, blk, W), jnp.float32),
            pltpu.SemaphoreType.DMA,
            pltpu.SemaphoreType.DMA,
        ],
    )(node_features)


def _sc_segsum(node_features, norms, seg_chunks):
    n = node_features.shape[0]
    rows_per_w = n // NW
    n_chunks = rows_per_w // CHUNK
    mesh = plsc.VectorSubcoreMesh(core_axis_name="c", subcore_axis_name="s")

    @functools.partial(
        pl.kernel,
        out_type=(
            jax.ShapeDtypeStruct((2 * NW, SEG, HS), jnp.float32),
            jax.ShapeDtypeStruct((2 * NW, SEG, HV), jnp.float32),
        ),
        mesh=mesh,
        compiler_params=pltpu.CompilerParams(needs_layout_passes=False),
        scratch_types=[
            pltpu.VMEM((CHUNK, HS), jnp.float32),
            pltpu.VMEM((CHUNK, HS), jnp.float32),
            pltpu.VMEM((CHUNK, HV), jnp.float32),
            pltpu.VMEM((CHUNK, HV), jnp.float32),
            pltpu.VMEM((n_chunks, CHUNK), jnp.int32),
            pltpu.VMEM((SEG, HS), jnp.float32),
            pltpu.VMEM_SHARED((2 * NSUB * SEG, HS), jnp.float32),
            pltpu.VMEM_SHARED((2 * NSUB * SEG, HV), jnp.float32),
            pltpu.SemaphoreType.DMA,
            pltpu.SemaphoreType.DMA,
            pltpu.SemaphoreType.DMA,
            pltpu.SemaphoreType.DMA,
            pltpu.SemaphoreType.DMA,
            pltpu.SemaphoreType.DMA,
            pltpu.SemaphoreType.DMA,
            pltpu.SemaphoreType.DMA,
        ],
    )
    def k(feat_hbm, norm_hbm, idx_hbm, outs_hbm, outn_hbm,
          bs0, bs1, bn0, bn1, ibuf, zbuf, acc_s, acc_n,
          sis0, sis1, sin0, sin1, sas0, sas1, san0, san1):
        cid = lax.axis_index("c")
        sid = lax.axis_index("s")
        wid = sid * NC + cid
        base = wid * rows_per_w
        sbufs, ssis, ssas = (bs0, bs1), (sis0, sis1), (sas0, sas1)
        nbufs, nsis, nsas = (bn0, bn1), (sin0, sin1), (san0, san1)

        def src_s(c):
            rb = base + c * CHUNK
            return feat_hbm.at[pl.ds(rb, CHUNK), pl.ds(0, HS)]

        def src_n(c):
            rb = base + c * CHUNK
            return norm_hbm.at[pl.ds(rb, CHUNK), :]

        pltpu.async_copy(src_s(0), bs0, sis0)
        pltpu.async_copy(src_n(0), bn0, sin0)
        pltpu.async_copy(src_s(1), bs1, sis1)
        pltpu.async_copy(src_n(1), bn1, sin1)

        pltpu.sync_copy(idx_hbm.at[wid], ibuf)
        for cc in range(n_chunks):
            off = ((2 * sid + (cc % 2)) * SEG).astype(jnp.int32)
            for j in range(CHUNK // L):
                ibuf[cc, pl.ds(j * L, L)] = ibuf[cc, pl.ds(j * L, L)] + off

        zeros = jnp.zeros((L,), jnp.float32)
        for s in range(SEG):
            for j in range(HS // L):
                zbuf[s, pl.ds(j * L, L)] = zeros
        pltpu.sync_copy(zbuf, acc_s.at[pl.ds(2 * sid * SEG, SEG)])
        pltpu.sync_copy(zbuf, acc_s.at[pl.ds((2 * sid + 1) * SEG, SEG)])
        pltpu.sync_copy(zbuf, acc_n.at[pl.ds(2 * sid * SEG, SEG)])
        pltpu.sync_copy(zbuf, acc_n.at[pl.ds((2 * sid + 1) * SEG, SEG)])

        for c in range(n_chunks):
            slot = c % 2
            idx = ibuf.at[c]
            pltpu.make_async_copy(src_s(c), sbufs[slot], ssis[slot]).wait()
            adds = pltpu.async_copy(
                sbufs[slot], acc_s.at[idx], ssas[slot], add=True)
            pltpu.make_async_copy(src_n(c), nbufs[slot], nsis[slot]).wait()
            addn = pltpu.async_copy(
                nbufs[slot], acc_n.at[idx], nsas[slot], add=True)
            adds.wait()
            if c + 2 < n_chunks:
                pltpu.async_copy(src_s(c + 2), sbufs[slot], ssis[slot])
            addn.wait()
            if c + 2 < n_chunks:
                pltpu.async_copy(src_n(c + 2), nbufs[slot], nsis[slot])

        pltpu.sync_copy(acc_s.at[pl.ds(2 * sid * SEG, SEG)],
                        outs_hbm.at[2 * wid])
        pltpu.sync_copy(acc_s.at[pl.ds((2 * sid + 1) * SEG, SEG)],
                        outs_hbm.at[2 * wid + 1])
        pltpu.sync_copy(acc_n.at[pl.ds(2 * sid * SEG, SEG)],
                        outn_hbm.at[2 * wid])
        pltpu.sync_copy(acc_n.at[pl.ds((2 * sid + 1) * SEG, SEG)],
                        outn_hbm.at[2 * wid + 1])

    return k(node_features, norms, seg_chunks)


def _tc_finalize(part_s, part_n, seg_ids_2d):
    def body(ps_ref, pn_ref, idx_ref, out_ref):
        ssum = jnp.sum(ps_ref[...], axis=0)
        nsum = jnp.sum(pn_ref[...], axis=0)
        b = idx_ref[...]
        counts = []
        for s in range(SEG):
            counts.append(jnp.sum(jnp.where(b == s, 1.0, 0.0)))
        cnt = jnp.maximum(jnp.stack(counts), 1.0)[:, None]
        out_ref[...] = jnp.concatenate([ssum, nsum], axis=-1) / cnt

    return pl.pallas_call(
        body,
        out_shape=jax.ShapeDtypeStruct((SEG, OUT_F), jnp.float32),
    )(part_s, part_n, seg_ids_2d)


def kernel(node_features, batch_idx, num_samples):
    n = batch_idx.shape[0]
    seg_ids = (batch_idx + (num_samples - SEG)).astype(jnp.int32)
    seg_chunks = seg_ids.reshape(NW, n // (NW * CHUNK), CHUNK)
    norms = _tc_norms(node_features)
    part_s, part_n = _sc_segsum(node_features, norms, seg_chunks)
    return _tc_finalize(part_s, part_n, seg_ids.reshape(n // 128, 128))


# speedup vs baseline: 1.0311x; 1.0311x over previous
import functools

import jax
import jax.numpy as jnp
from jax import lax
from jax.experimental import pallas as pl
from jax.experimental.pallas import tpu as pltpu
from jax.experimental.pallas import tpu_sc as plsc

HS = 128
HV = 128
FEAT = HS + 3 * HV
SEG = 16
OUT_F = HS + HV
NC, NSUB, L = 2, 16, 16
NW = NC * NSUB
CHUNK = 128


def _tc_norms(node_features):
    n = node_features.shape[0]
    blk = 2048
    steps = n // blk
    W = 3 * HV

    def body(hbm_ref, o_ref, buf, s0, s1):
        i = pl.program_id(0)

        def src(step):
            return hbm_ref.at[pl.ds(step * blk, blk), pl.ds(HS, W)]

        @pl.when(i == 0)
        def _():
            pltpu.async_copy(src(0), buf.at[0], s0)

        nxt = i + 1

        @pl.when(jnp.logical_and(nxt < steps, nxt % 2 == 0))
        def _():
            pltpu.async_copy(src(nxt), buf.at[0], s0)

        @pl.when(jnp.logical_and(nxt < steps, nxt % 2 == 1))
        def _():
            pltpu.async_copy(src(nxt), buf.at[1], s1)

        @pl.when(i % 2 == 0)
        def _():
            pltpu.make_async_copy(src(i), buf.at[0], s0).wait()

        @pl.when(i % 2 == 1)
        def _():
            pltpu.make_async_copy(src(i), buf.at[1], s1).wait()

        v = buf[i % 2]
        sq = v * v
        r = lax.broadcasted_iota(jnp.int32, (W, HV), 0)
        c = lax.broadcasted_iota(jnp.int32, (W, HV), 1)
        sel = jnp.where(r // 3 == c, 1.0, 0.0)
        ss = jnp.dot(sq, sel, preferred_element_type=jnp.float32)
        o_ref[...] = jnp.sqrt(ss)

    return pl.pallas_call(
        body,
        grid=(steps,),
        in_specs=[pl.BlockSpec(memory_space=pltpu.MemorySpace.HBM)],
        out_specs=pl.BlockSpec((blk, HV), lambda i: (i, 0)),
        out_shape=jax.ShapeDtypeStruct((n, HV), jnp.float32),
        scratch_shapes=[
            pltpu.VMEM((2, blk, W), jnp.float32),
            pltpu.SemaphoreType.DMA,
            pltpu.SemaphoreType.DMA,
        ],
    )(node_features)


def _sc_segsum(node_features, norms, seg_chunks):
    n = node_features.shape[0]
    rows_per_w = n // NW
    n_chunks = rows_per_w // CHUNK
    mesh = plsc.VectorSubcoreMesh(core_axis_name="c", subcore_axis_name="s")

    @functools.partial(
        pl.kernel,
        out_type=(
            jax.ShapeDtypeStruct((NW, SEG, HS), jnp.float32),
            jax.ShapeDtypeStruct((NW, SEG, HV), jnp.float32),
        ),
        mesh=mesh,
        compiler_params=pltpu.CompilerParams(needs_layout_passes=False),
        scratch_types=[
            pltpu.VMEM((CHUNK, HS), jnp.float32),
            pltpu.VMEM((CHUNK, HS), jnp.float32),
            pltpu.VMEM((CHUNK, HV), jnp.float32),
            pltpu.VMEM((CHUNK, HV), jnp.float32),
            pltpu.VMEM((n_chunks, CHUNK), jnp.int32),
            pltpu.VMEM((SEG, HS), jnp.float32),
            pltpu.VMEM_SHARED((NSUB * SEG, HS), jnp.float32),
            pltpu.VMEM_SHARED((NSUB * SEG, HV), jnp.float32),
            pltpu.SemaphoreType.DMA,
            pltpu.SemaphoreType.DMA,
            pltpu.SemaphoreType.DMA,
            pltpu.SemaphoreType.DMA,
            pltpu.SemaphoreType.DMA,
            pltpu.SemaphoreType.DMA,
            pltpu.SemaphoreType.DMA,
            pltpu.SemaphoreType.DMA,
        ],
    )
    def k(feat_hbm, norm_hbm, idx_hbm, outs_hbm, outn_hbm,
          bs0, bs1, bn0, bn1, ibuf, zbuf, acc_s, acc_n,
          sis0, sis1, sin0, sin1, sas0, sas1, san0, san1):
        cid = lax.axis_index("c")
        sid = lax.axis_index("s")
        wid = sid * NC + cid
        base = wid * rows_per_w
        sbufs, ssis, ssas = (bs0, bs1), (sis0, sis1), (sas0, sas1)
        nbufs, nsis, nsas = (bn0, bn1), (sin0, sin1), (san0, san1)

        def src_s(c):
            rb = base + c * CHUNK
            return feat_hbm.at[pl.ds(rb, CHUNK), pl.ds(0, HS)]

        def src_n(c):
            rb = base + c * CHUNK
            return norm_hbm.at[pl.ds(rb, CHUNK), :]

        pltpu.async_copy(src_s(0), bs0, sis0)
        pltpu.async_copy(src_n(0), bn0, sin0)
        pltpu.async_copy(src_s(1), bs1, sis1)
        pltpu.async_copy(src_n(1), bn1, sin1)

        pltpu.sync_copy(idx_hbm.at[wid], ibuf)
        off = (sid * SEG).astype(jnp.int32)
        for cc in range(n_chunks):
            for j in range(CHUNK // L):
                ibuf[cc, pl.ds(j * L, L)] = ibuf[cc, pl.ds(j * L, L)] + off

        zeros = jnp.zeros((L,), jnp.float32)
        for s in range(SEG):
            for j in range(HS // L):
                zbuf[s, pl.ds(j * L, L)] = zeros
        pltpu.sync_copy(zbuf, acc_s.at[pl.ds(sid * SEG, SEG)])
        pltpu.sync_copy(zbuf, acc_n.at[pl.ds(sid * SEG, SEG)])

        for c in range(n_chunks):
            slot = c % 2
            idx = ibuf.at[c]
            pltpu.make_async_copy(src_s(c), sbufs[slot], ssis[slot]).wait()
            adds = pltpu.async_copy(
                sbufs[slot], acc_s.at[idx], ssas[slot], add=True)
            pltpu.make_async_copy(src_n(c), nbufs[slot], nsis[slot]).wait()
            addn = pltpu.async_copy(
                nbufs[slot], acc_n.at[idx], nsas[slot], add=True)
            adds.wait()
            if c + 2 < n_chunks:
                pltpu.async_copy(src_s(c + 2), sbufs[slot], ssis[slot])
            addn.wait()
            if c + 2 < n_chunks:
                pltpu.async_copy(src_n(c + 2), nbufs[slot], nsis[slot])

        pltpu.sync_copy(acc_s.at[pl.ds(sid * SEG, SEG)], outs_hbm.at[wid])
        pltpu.sync_copy(acc_n.at[pl.ds(sid * SEG, SEG)], outn_hbm.at[wid])

    return k(node_features, norms, seg_chunks)


def _tc_finalize(part_s, part_n, seg_ids_2d):
    def body(ps_ref, pn_ref, idx_ref, out_ref):
        ssum = jnp.sum(ps_ref[...], axis=0)
        nsum = jnp.sum(pn_ref[...], axis=0)
        b = idx_ref[...]
        counts = []
        for s in range(SEG):
            counts.append(jnp.sum(jnp.where(b == s, 1.0, 0.0)))
        cnt = jnp.maximum(jnp.stack(counts), 1.0)[:, None]
        out_ref[...] = jnp.concatenate([ssum, nsum], axis=-1) / cnt

    return pl.pallas_call(
        body,
        out_shape=jax.ShapeDtypeStruct((SEG, OUT_F), jnp.float32),
    )(part_s, part_n, seg_ids_2d)


def kernel(node_features, batch_idx, num_samples):
    n = batch_idx.shape[0]
    seg_ids = (batch_idx + (num_samples - SEG)).astype(jnp.int32)
    seg_chunks = seg_ids.reshape(NW, n // (NW * CHUNK), CHUNK)
    norms = _tc_norms(node_features)
    part_s, part_n = _sc_segsum(node_features, norms, seg_chunks)
    return _tc_finalize(part_s, part_n, seg_ids.reshape(n // 128, 128))


# speedup vs baseline: 1.0728x; 1.0404x over previous
import functools

import jax
import jax.numpy as jnp
from jax import lax
from jax.experimental import pallas as pl
from jax.experimental.pallas import tpu as pltpu
from jax.experimental.pallas import tpu_sc as plsc

HS = 128
HV = 128
FEAT = HS + 3 * HV
SEG = 16
OUT_F = HS + HV
NC, NSUB, L = 2, 16, 16
NW = NC * NSUB
CHUNK = 128


def _tc_norms(node_features):
    n = node_features.shape[0]
    blk = 4096
    steps = n // blk
    W = 3 * HV

    def body(hbm_ref, o_ref, buf, s0, s1):
        i = pl.program_id(0)

        def src(step):
            return hbm_ref.at[pl.ds(step * blk, blk), pl.ds(HS, W)]

        @pl.when(i == 0)
        def _():
            pltpu.async_copy(src(0), buf.at[0], s0)

        nxt = i + 1

        @pl.when(jnp.logical_and(nxt < steps, nxt % 2 == 0))
        def _():
            pltpu.async_copy(src(nxt), buf.at[0], s0)

        @pl.when(jnp.logical_and(nxt < steps, nxt % 2 == 1))
        def _():
            pltpu.async_copy(src(nxt), buf.at[1], s1)

        @pl.when(i % 2 == 0)
        def _():
            pltpu.make_async_copy(src(i), buf.at[0], s0).wait()

        @pl.when(i % 2 == 1)
        def _():
            pltpu.make_async_copy(src(i), buf.at[1], s1).wait()

        v = buf[i % 2]
        sq = v * v
        r = lax.broadcasted_iota(jnp.int32, (W, HV), 0)
        c = lax.broadcasted_iota(jnp.int32, (W, HV), 1)
        sel = jnp.where(r // 3 == c, 1.0, 0.0)
        ss = jnp.dot(sq, sel, preferred_element_type=jnp.float32)
        o_ref[...] = jnp.sqrt(ss)

    return pl.pallas_call(
        body,
        grid=(steps,),
        in_specs=[pl.BlockSpec(memory_space=pltpu.MemorySpace.HBM)],
        out_specs=pl.BlockSpec((blk, HV), lambda i: (i, 0)),
        out_shape=jax.ShapeDtypeStruct((n, HV), jnp.float32),
        scratch_shapes=[
            pltpu.VMEM((2, blk, W), jnp.float32),
            pltpu.SemaphoreType.DMA,
            pltpu.SemaphoreType.DMA,
        ],
    )(node_features)


def _sc_segsum(node_features, norms, seg_chunks):
    n = node_features.shape[0]
    rows_per_w = n // NW
    n_chunks = rows_per_w // CHUNK
    mesh = plsc.VectorSubcoreMesh(core_axis_name="c", subcore_axis_name="s")

    @functools.partial(
        pl.kernel,
        out_type=(
            jax.ShapeDtypeStruct((NW, SEG, HS), jnp.float32),
            jax.ShapeDtypeStruct((NW, SEG, HV), jnp.float32),
        ),
        mesh=mesh,
        compiler_params=pltpu.CompilerParams(needs_layout_passes=False),
        scratch_types=[
            pltpu.VMEM((CHUNK, HS), jnp.float32),
            pltpu.VMEM((CHUNK, HS), jnp.float32),
            pltpu.VMEM((CHUNK, HV), jnp.float32),
            pltpu.VMEM((CHUNK, HV), jnp.float32),
            pltpu.VMEM((n_chunks, CHUNK), jnp.int32),
            pltpu.VMEM((SEG, HS), jnp.float32),
            pltpu.VMEM_SHARED((NSUB * SEG, HS), jnp.float32),
            pltpu.VMEM_SHARED((NSUB * SEG, HV), jnp.float32),
            pltpu.SemaphoreType.DMA,
            pltpu.SemaphoreType.DMA,
            pltpu.SemaphoreType.DMA,
            pltpu.SemaphoreType.DMA,
            pltpu.SemaphoreType.DMA,
            pltpu.SemaphoreType.DMA,
            pltpu.SemaphoreType.DMA,
            pltpu.SemaphoreType.DMA,
        ],
    )
    def k(feat_hbm, norm_hbm, idx_hbm, outs_hbm, outn_hbm,
          bs0, bs1, bn0, bn1, ibuf, zbuf, acc_s, acc_n,
          sis0, sis1, sin0, sin1, sas0, sas1, san0, san1):
        cid = lax.axis_index("c")
        sid = lax.axis_index("s")
        wid = sid * NC + cid
        base = wid * rows_per_w
        sbufs, ssis, ssas = (bs0, bs1), (sis0, sis1), (sas0, sas1)
        nbufs, nsis, nsas = (bn0, bn1), (sin0, sin1), (san0, san1)

        def src_s(c):
            rb = base + c * CHUNK
            return feat_hbm.at[pl.ds(rb, CHUNK), pl.ds(0, HS)]

        def src_n(c):
            rb = base + c * CHUNK
            return norm_hbm.at[pl.ds(rb, CHUNK), :]

        pltpu.async_copy(src_s(0), bs0, sis0)
        pltpu.async_copy(src_n(0), bn0, sin0)
        pltpu.async_copy(src_s(1), bs1, sis1)
        pltpu.async_copy(src_n(1), bn1, sin1)

        pltpu.sync_copy(idx_hbm.at[wid], ibuf)
        off = (sid * SEG).astype(jnp.int32)
        for cc in range(n_chunks):
            for j in range(CHUNK // L):
                ibuf[cc, pl.ds(j * L, L)] = ibuf[cc, pl.ds(j * L, L)] + off

        zeros = jnp.zeros((L,), jnp.float32)
        for s in range(SEG):
            for j in range(HS // L):
                zbuf[s, pl.ds(j * L, L)] = zeros
        pltpu.sync_copy(zbuf, acc_s.at[pl.ds(sid * SEG, SEG)])
        pltpu.sync_copy(zbuf, acc_n.at[pl.ds(sid * SEG, SEG)])

        for c in range(n_chunks):
            slot = c % 2
            idx = ibuf.at[c]
            pltpu.make_async_copy(src_s(c), sbufs[slot], ssis[slot]).wait()
            adds = pltpu.async_copy(
                sbufs[slot], acc_s.at[idx], ssas[slot], add=True)
            pltpu.make_async_copy(src_n(c), nbufs[slot], nsis[slot]).wait()
            addn = pltpu.async_copy(
                nbufs[slot], acc_n.at[idx], nsas[slot], add=True)
            adds.wait()
            if c + 2 < n_chunks:
                pltpu.async_copy(src_s(c + 2), sbufs[slot], ssis[slot])
            addn.wait()
            if c + 2 < n_chunks:
                pltpu.async_copy(src_n(c + 2), nbufs[slot], nsis[slot])

        pltpu.sync_copy(acc_s.at[pl.ds(sid * SEG, SEG)], outs_hbm.at[wid])
        pltpu.sync_copy(acc_n.at[pl.ds(sid * SEG, SEG)], outn_hbm.at[wid])

    return k(node_features, norms, seg_chunks)


def _tc_finalize(part_s, part_n, seg_ids_2d):
    def body(ps_ref, pn_ref, idx_ref, out_ref):
        ssum = jnp.sum(ps_ref[...], axis=0)
        nsum = jnp.sum(pn_ref[...], axis=0)
        b = idx_ref[...]
        counts = []
        for s in range(SEG):
            counts.append(jnp.sum(jnp.where(b == s, 1.0, 0.0)))
        cnt = jnp.maximum(jnp.stack(counts), 1.0)[:, None]
        out_ref[...] = jnp.concatenate([ssum, nsum], axis=-1) / cnt

    return pl.pallas_call(
        body,
        out_shape=jax.ShapeDtypeStruct((SEG, OUT_F), jnp.float32),
    )(part_s, part_n, seg_ids_2d)


def kernel(node_features, batch_idx, num_samples):
    n = batch_idx.shape[0]
    seg_ids = (batch_idx + (num_samples - SEG)).astype(jnp.int32)
    seg_chunks = seg_ids.reshape(NW, n // (NW * CHUNK), CHUNK)
    norms = _tc_norms(node_features)
    part_s, part_n = _sc_segsum(node_features, norms, seg_chunks)
    return _tc_finalize(part_s, part_n, seg_ids.reshape(n // 128, 128))


# speedup vs baseline: 1.0841x; 1.0105x over previous
import functools

import jax
import jax.numpy as jnp
from jax import lax
from jax.experimental import pallas as pl
from jax.experimental.pallas import tpu as pltpu
from jax.experimental.pallas import tpu_sc as plsc

HS = 128
HV = 128
FEAT = HS + 3 * HV
SEG = 16
OUT_F = HS + HV
NC, NSUB, L = 2, 16, 16
NW = NC * NSUB
CHUNK = 128


def _tc_norms(node_features):
    n = node_features.shape[0]
    blk = 8192
    steps = n // blk
    W = 3 * HV

    def body(hbm_ref, o_ref, buf, s0, s1):
        i = pl.program_id(0)

        def src(step):
            return hbm_ref.at[pl.ds(step * blk, blk), pl.ds(HS, W)]

        @pl.when(i == 0)
        def _():
            pltpu.async_copy(src(0), buf.at[0], s0)

        nxt = i + 1

        @pl.when(jnp.logical_and(nxt < steps, nxt % 2 == 0))
        def _():
            pltpu.async_copy(src(nxt), buf.at[0], s0)

        @pl.when(jnp.logical_and(nxt < steps, nxt % 2 == 1))
        def _():
            pltpu.async_copy(src(nxt), buf.at[1], s1)

        @pl.when(i % 2 == 0)
        def _():
            pltpu.make_async_copy(src(i), buf.at[0], s0).wait()

        @pl.when(i % 2 == 1)
        def _():
            pltpu.make_async_copy(src(i), buf.at[1], s1).wait()

        v = buf[i % 2]
        sq = v * v
        r = lax.broadcasted_iota(jnp.int32, (W, HV), 0)
        c = lax.broadcasted_iota(jnp.int32, (W, HV), 1)
        sel = jnp.where(r // 3 == c, 1.0, 0.0)
        ss = jnp.dot(sq, sel, preferred_element_type=jnp.float32)
        o_ref[...] = jnp.sqrt(ss)

    return pl.pallas_call(
        body,
        grid=(steps,),
        in_specs=[pl.BlockSpec(memory_space=pltpu.MemorySpace.HBM)],
        out_specs=pl.BlockSpec((blk, HV), lambda i: (i, 0)),
        out_shape=jax.ShapeDtypeStruct((n, HV), jnp.float32),
        scratch_shapes=[
            pltpu.VMEM((2, blk, W), jnp.float32),
            pltpu.SemaphoreType.DMA,
            pltpu.SemaphoreType.DMA,
        ],
    )(node_features)


def _sc_segsum(node_features, norms, seg_chunks):
    n = node_features.shape[0]
    rows_per_w = n // NW
    n_chunks = rows_per_w // CHUNK
    mesh = plsc.VectorSubcoreMesh(core_axis_name="c", subcore_axis_name="s")

    @functools.partial(
        pl.kernel,
        out_type=(
            jax.ShapeDtypeStruct((NW, SEG, HS), jnp.float32),
            jax.ShapeDtypeStruct((NW, SEG, HV), jnp.float32),
        ),
        mesh=mesh,
        compiler_params=pltpu.CompilerParams(needs_layout_passes=False),
        scratch_types=[
            pltpu.VMEM((CHUNK, HS), jnp.float32),
            pltpu.VMEM((CHUNK, HS), jnp.float32),
            pltpu.VMEM((CHUNK, HV), jnp.float32),
            pltpu.VMEM((CHUNK, HV), jnp.float32),
            pltpu.VMEM((n_chunks, CHUNK), jnp.int32),
            pltpu.VMEM((SEG, HS), jnp.float32),
            pltpu.VMEM_SHARED((NSUB * SEG, HS), jnp.float32),
            pltpu.VMEM_SHARED((NSUB * SEG, HV), jnp.float32),
            pltpu.SemaphoreType.DMA,
            pltpu.SemaphoreType.DMA,
            pltpu.SemaphoreType.DMA,
            pltpu.SemaphoreType.DMA,
            pltpu.SemaphoreType.DMA,
            pltpu.SemaphoreType.DMA,
            pltpu.SemaphoreType.DMA,
            pltpu.SemaphoreType.DMA,
        ],
    )
    def k(feat_hbm, norm_hbm, idx_hbm, outs_hbm, outn_hbm,
          bs0, bs1, bn0, bn1, ibuf, zbuf, acc_s, acc_n,
          sis0, sis1, sin0, sin1, sas0, sas1, san0, san1):
        cid = lax.axis_index("c")
        sid = lax.axis_index("s")
        wid = sid * NC + cid
        base = wid * rows_per_w
        sbufs, ssis, ssas = (bs0, bs1), (sis0, sis1), (sas0, sas1)
        nbufs, nsis, nsas = (bn0, bn1), (sin0, sin1), (san0, san1)

        def src_s(c):
            rb = base + c * CHUNK
            return feat_hbm.at[pl.ds(rb, CHUNK), pl.ds(0, HS)]

        def src_n(c):
            rb = base + c * CHUNK
            return norm_hbm.at[pl.ds(rb, CHUNK), :]

        pltpu.async_copy(src_s(0), bs0, sis0)
        pltpu.async_copy(src_n(0), bn0, sin0)
        pltpu.async_copy(src_s(1), bs1, sis1)
        pltpu.async_copy(src_n(1), bn1, sin1)

        pltpu.sync_copy(idx_hbm.at[wid], ibuf)
        off = (sid * SEG).astype(jnp.int32)
        for cc in range(n_chunks):
            for j in range(CHUNK // L):
                ibuf[cc, pl.ds(j * L, L)] = ibuf[cc, pl.ds(j * L, L)] + off

        zeros = jnp.zeros((L,), jnp.float32)
        for s in range(SEG):
            for j in range(HS // L):
                zbuf[s, pl.ds(j * L, L)] = zeros
        pltpu.sync_copy(zbuf, acc_s.at[pl.ds(sid * SEG, SEG)])
        pltpu.sync_copy(zbuf, acc_n.at[pl.ds(sid * SEG, SEG)])

        for c in range(n_chunks):
            slot = c % 2
            idx = ibuf.at[c]
            pltpu.make_async_copy(src_s(c), sbufs[slot], ssis[slot]).wait()
            adds = pltpu.async_copy(
                sbufs[slot], acc_s.at[idx], ssas[slot], add=True)
            pltpu.make_async_copy(src_n(c), nbufs[slot], nsis[slot]).wait()
            addn = pltpu.async_copy(
                nbufs[slot], acc_n.at[idx], nsas[slot], add=True)
            adds.wait()
            if c + 2 < n_chunks:
                pltpu.async_copy(src_s(c + 2), sbufs[slot], ssis[slot])
            addn.wait()
            if c + 2 < n_chunks:
                pltpu.async_copy(src_n(c + 2), nbufs[slot], nsis[slot])

        pltpu.sync_copy(acc_s.at[pl.ds(sid * SEG, SEG)], outs_hbm.at[wid])
        pltpu.sync_copy(acc_n.at[pl.ds(sid * SEG, SEG)], outn_hbm.at[wid])

    return k(node_features, norms, seg_chunks)


def _tc_finalize(part_s, part_n, seg_ids_2d):
    def body(ps_ref, pn_ref, idx_ref, out_ref):
        ssum = jnp.sum(ps_ref[...], axis=0)
        nsum = jnp.sum(pn_ref[...], axis=0)
        b = idx_ref[...]
        counts = []
        for s in range(SEG):
            counts.append(jnp.sum(jnp.where(b == s, 1.0, 0.0)))
        cnt = jnp.maximum(jnp.stack(counts), 1.0)[:, None]
        out_ref[...] = jnp.concatenate([ssum, nsum], axis=-1) / cnt

    return pl.pallas_call(
        body,
        out_shape=jax.ShapeDtypeStruct((SEG, OUT_F), jnp.float32),
    )(part_s, part_n, seg_ids_2d)


def kernel(node_features, batch_idx, num_samples):
    n = batch_idx.shape[0]
    seg_ids = (batch_idx + (num_samples - SEG)).astype(jnp.int32)
    seg_chunks = seg_ids.reshape(NW, n // (NW * CHUNK), CHUNK)
    norms = _tc_norms(node_features)
    part_s, part_n = _sc_segsum(node_features, norms, seg_chunks)
    return _tc_finalize(part_s, part_n, seg_ids.reshape(n // 128, 128))
